# Initial kernel scaffold; baseline (speedup 1.0000x reference)
import jax
import jax.numpy as jnp
from jax.experimental import pallas as pl


def kernel(atype, weight):
    raise NotImplementedError("write your pallas kernel here")


# speedup vs baseline: 5.7032x; 5.7032x over previous
import functools

import jax
import jax.numpy as jnp
from jax import lax
from jax.experimental import pallas as pl
from jax.experimental.pallas import tpu as pltpu
from jax.experimental.pallas import tpu_sc as plsc

NC = 2
NS = 16
NW = NC * NS

B = 4096 * 200
D = 128
CH = 128
BPW = B // NW
NCH = BPW // CH
K = 5
NGRP = NCH // K

_mesh = plsc.VectorSubcoreMesh(
    core_axis_name="c", subcore_axis_name="s", num_cores=NC, num_subcores=NS
)


@functools.partial(
    pl.kernel,
    mesh=_mesh,
    out_type=jax.ShapeDtypeStruct((B, D), jnp.float32),
    scratch_types=[
        pltpu.VMEM((NCH, CH), jnp.int32),
        pltpu.VMEM((K * CH, D), jnp.float32),
        pltpu.SemaphoreType.DMA,
    ],
)
def _embed_sc(idx_hbm, table_hbm, out_hbm, idx_v, rows_v, gsem):
    wid = lax.axis_index("s") * NC + lax.axis_index("c")
    base = wid * BPW

    pltpu.sync_copy(idx_hbm.at[pl.ds(wid * NCH, NCH)], idx_v)

    def group(g, carry):
        copies = [
            pltpu.async_copy(
                table_hbm.at[idx_v.at[g * K + b]],
                rows_v.at[pl.ds(b * CH, CH)],
                gsem,
            )
            for b in range(K)
        ]
        for cp in copies:
            cp.wait()
        pltpu.sync_copy(rows_v, out_hbm.at[pl.ds(base + g * (K * CH), K * CH)])
        return carry

    lax.fori_loop(0, NGRP, group, 0)


def kernel(atype, weight):
    idx2d = atype.reshape(B // CH, CH)
    out = _embed_sc(idx2d, weight)
    return out.reshape(atype.shape[0], atype.shape[1], D)


# speedup vs baseline: 5.8026x; 1.0174x over previous
import functools

import jax
import jax.numpy as jnp
from jax import lax
from jax.experimental import pallas as pl
from jax.experimental.pallas import tpu as pltpu
from jax.experimental.pallas import tpu_sc as plsc

NC = 2
NS = 16
NW = NC * NS

B = 4096 * 200
D = 128
CH = 100
G = 2
GRP = G * CH
BPW = B // NW
NCH = BPW // CH
NGRP = BPW // GRP
NBUF = 3

_mesh = plsc.VectorSubcoreMesh(
    core_axis_name="c", subcore_axis_name="s", num_cores=NC, num_subcores=NS
)


@functools.partial(
    pl.kernel,
    mesh=_mesh,
    out_type=jax.ShapeDtypeStruct((B, D), jnp.float32),
    scratch_types=[
        pltpu.VMEM((NCH, CH), jnp.int32),
        pltpu.VMEM((GRP, D), jnp.float32),
        pltpu.VMEM((GRP, D), jnp.float32),
        pltpu.VMEM((GRP, D), jnp.float32),
        pltpu.SemaphoreType.DMA,
        pltpu.SemaphoreType.DMA,
        pltpu.SemaphoreType.DMA,
        pltpu.SemaphoreType.DMA,
        pltpu.SemaphoreType.DMA,
        pltpu.SemaphoreType.DMA,
    ],
)
def _embed_sc(idx_hbm, table_hbm, out_hbm, idx_v, r0, r1, r2,
              g0, g1, g2, s0, s1, s2):
    rows = (r0, r1, r2)
    gsem = (g0, g1, g2)
    ssem = (s0, s1, s2)
    wid = lax.axis_index("s") * NC + lax.axis_index("c")
    base = wid * BPW

    pltpu.sync_copy(idx_hbm.at[pl.ds(wid * NCH, NCH)], idx_v)

    def fire_gathers(g, buf):
        for b in range(G):
            pltpu.async_copy(
                table_hbm.at[idx_v.at[g * G + b]],
                rows[buf].at[pl.ds(b * CH, CH)],
                gsem[buf],
            )

    def step(g, cur, wait_prev, fire_next):
        prev = (cur - 1) % NBUF
        for b in range(G):
            pltpu.make_async_copy(
                table_hbm.at[idx_v.at[b]],
                rows[cur].at[pl.ds(b * CH, CH)],
                gsem[cur],
            ).wait()
        pltpu.async_copy(
            rows[cur], out_hbm.at[pl.ds(base + g * GRP, GRP)], ssem[cur]
        )
        if wait_prev:
            pltpu.make_async_copy(
                rows[prev], out_hbm.at[pl.ds(base, GRP)], ssem[prev]
            ).wait()
        if fire_next:
            fire_gathers(g + 2, prev)

    fire_gathers(0, 0)
    fire_gathers(1, 1)

    step(0, 0, wait_prev=False, fire_next=True)

    def body(t, carry):
        for b in range(NBUF):
            g = 1 + t * NBUF + b
            step(g, (1 + b) % NBUF, wait_prev=True, fire_next=True)
        return carry

    lax.fori_loop(0, (NGRP - 5) // NBUF, body, 0)

    for g in (NGRP - 4, NGRP - 3):
        step(g, g % NBUF, wait_prev=True, fire_next=True)
    for g in (NGRP - 2, NGRP - 1):
        step(g, g % NBUF, wait_prev=True, fire_next=False)

    pltpu.make_async_copy(
        rows[(NGRP - 1) % NBUF], out_hbm.at[pl.ds(base, GRP)],
        ssem[(NGRP - 1) % NBUF],
    ).wait()


def kernel(atype, weight):
    idx2d = atype.reshape(B // CH, CH)
    out = _embed_sc(idx2d, weight)
    return out.reshape(atype.shape[0], atype.shape[1], D)


# speedup vs baseline: 15.4786x; 2.6675x over previous
import functools

import jax
import jax.numpy as jnp
from jax import lax
from jax.experimental import pallas as pl
from jax.experimental.pallas import tpu as pltpu
from jax.experimental.pallas import tpu_sc as plsc

NC = 2
NS = 16
NW = NC * NS

B = 4096 * 200
D = 128
CH = 100
G = 2
GRP = G * CH
BPW = B // NW
NCH = BPW // CH
NGRP = BPW // GRP
NBUF = 3

_mesh = plsc.VectorSubcoreMesh(
    core_axis_name="c", subcore_axis_name="s", num_cores=NC, num_subcores=NS
)


@functools.partial(
    pl.kernel,
    mesh=_mesh,
    out_type=jax.ShapeDtypeStruct((B, D), jnp.float32),
    scratch_types=[
        pltpu.VMEM((NCH, CH), jnp.int32),
        pltpu.VMEM((GRP, D), jnp.float32),
        pltpu.VMEM((GRP, D), jnp.float32),
        pltpu.VMEM((GRP, D), jnp.float32),
        pltpu.VMEM_SHARED((1001, D), jnp.float32),
        pltpu.SemaphoreType.DMA,
        pltpu.SemaphoreType.DMA,
        pltpu.SemaphoreType.DMA,
        pltpu.SemaphoreType.DMA,
        pltpu.SemaphoreType.DMA,
        pltpu.SemaphoreType.DMA,
    ],
)
def _embed_sc(idx_hbm, table_hbm, out_hbm, idx_v, r0, r1, r2, table_sh,
              g0, g1, g2, s0, s1, s2):
    rows = (r0, r1, r2)
    gsem = (g0, g1, g2)
    ssem = (s0, s1, s2)
    sid = lax.axis_index("s")
    wid = sid * NC + lax.axis_index("c")
    base = wid * BPW

    @pl.when(sid == 0)
    def _():
        pltpu.sync_copy(table_hbm, table_sh)

    pltpu.sync_copy(idx_hbm.at[pl.ds(wid * NCH, NCH)], idx_v)
    plsc.subcore_barrier()

    def fire_gathers(g, buf):
        for b in range(G):
            pltpu.async_copy(
                table_sh.at[idx_v.at[g * G + b]],
                rows[buf].at[pl.ds(b * CH, CH)],
                gsem[buf],
            )

    def step(g, cur, wait_prev, fire_next):
        prev = (cur - 1) % NBUF
        for b in range(G):
            pltpu.make_async_copy(
                table_hbm.at[idx_v.at[b]],
                rows[cur].at[pl.ds(b * CH, CH)],
                gsem[cur],
            ).wait()
        pltpu.async_copy(
            rows[cur], out_hbm.at[pl.ds(base + g * GRP, GRP)], ssem[cur]
        )
        if wait_prev:
            pltpu.make_async_copy(
                rows[prev], out_hbm.at[pl.ds(base, GRP)], ssem[prev]
            ).wait()
        if fire_next:
            fire_gathers(g + 2, prev)

    fire_gathers(0, 0)
    fire_gathers(1, 1)

    step(0, 0, wait_prev=False, fire_next=True)

    def body(t, carry):
        for b in range(NBUF):
            g = 1 + t * NBUF + b
            step(g, (1 + b) % NBUF, wait_prev=True, fire_next=True)
        return carry

    lax.fori_loop(0, (NGRP - 5) // NBUF, body, 0)

    for g in (NGRP - 4, NGRP - 3):
        step(g, g % NBUF, wait_prev=True, fire_next=True)
    for g in (NGRP - 2, NGRP - 1):
        step(g, g % NBUF, wait_prev=True, fire_next=False)

    pltpu.make_async_copy(
        rows[(NGRP - 1) % NBUF], out_hbm.at[pl.ds(base, GRP)],
        ssem[(NGRP - 1) % NBUF],
    ).wait()


def kernel(atype, weight):
    idx2d = atype.reshape(B // CH, CH)
    out = _embed_sc(idx2d, weight)
    return out.reshape(atype.shape[0], atype.shape[1], D)
